# baseline (device time: 188547 ns/iter reference)
import jax
import jax.numpy as jnp
from jax import lax
from jax.experimental import pallas as pl
from jax.experimental.pallas import tpu as pltpu

N_DEV = 8


def kernel(x, w_mat):
    m, _ = x.shape
    kk, n = w_mat.shape
    m_per = m // N_DEV

    def body(x_ref, w_ref, out_ref, comm_ref, w16_ref, send_sems, recv_sems):
        my = lax.axis_index("i")
        left = lax.rem(my + N_DEV - 1, N_DEV)
        right = lax.rem(my + 1, N_DEV)

        barrier_sem = pltpu.get_barrier_semaphore()
        for nbr in (left, right):
            pl.semaphore_signal(
                barrier_sem, inc=1,
                device_id=(nbr,), device_id_type=pl.DeviceIdType.MESH,
            )
        pl.semaphore_wait(barrier_sem, 2)

        w16_ref[...] = w_ref[...].astype(jnp.bfloat16)

        def partial_chunk(c):
            xs = x_ref[pl.ds(c * m_per, m_per), :].astype(jnp.bfloat16)
            return jnp.dot(xs, w16_ref[...], preferred_element_type=jnp.float32)

        c0 = lax.rem(my + N_DEV - 1, N_DEV)
        comm_ref[0, :, :] = partial_chunk(c0).astype(jnp.bfloat16)

        for h in range(N_DEV - 1):
            rdma = pltpu.make_async_remote_copy(
                src_ref=comm_ref.at[h],
                dst_ref=comm_ref.at[h + 1],
                send_sem=send_sems.at[h],
                recv_sem=recv_sems.at[h],
                device_id=(right,),
                device_id_type=pl.DeviceIdType.MESH,
            )
            rdma.start()
            c = lax.rem(my + 2 * N_DEV - 2 - h, N_DEV)
            p = partial_chunk(c)
            rdma.wait()
            if h < N_DEV - 2:
                acc = comm_ref[h + 1, :, :].astype(jnp.float32) + p
                comm_ref[h + 1, :, :] = acc.astype(jnp.bfloat16)
            else:
                y = comm_ref[h + 1, :, :].astype(jnp.float32) + p
                out_ref[...] = y * jax.nn.sigmoid(y)

    return pl.pallas_call(
        body,
        out_shape=jax.ShapeDtypeStruct((m_per, n), jnp.float32),
        in_specs=[
            pl.BlockSpec(memory_space=pltpu.VMEM),
            pl.BlockSpec(memory_space=pltpu.VMEM),
        ],
        out_specs=pl.BlockSpec(memory_space=pltpu.VMEM),
        scratch_shapes=[
            pltpu.VMEM((N_DEV, m_per, n), jnp.bfloat16),
            pltpu.VMEM((kk, n), jnp.bfloat16),
            pltpu.SemaphoreType.DMA((N_DEV - 1,)),
            pltpu.SemaphoreType.DMA((N_DEV - 1,)),
        ],
        compiler_params=pltpu.CompilerParams(collective_id=0),
    )(x, w_mat)


# device time: 112753 ns/iter; 1.6722x vs baseline; 1.6722x over previous
import jax
import jax.numpy as jnp
from jax import lax
from jax.experimental import pallas as pl
from jax.experimental.pallas import tpu as pltpu

N_DEV = 8


def kernel(x, w_mat):
    m, _ = x.shape
    kk, n = w_mat.shape
    m_per = m // N_DEV
    n_half = n // 2

    def body(x_ref, w_ref, out_ref, comm_f, comm_b, w16_ref,
             send_f, recv_f, send_b, recv_b):
        my = lax.axis_index("i")
        left = lax.rem(my + N_DEV - 1, N_DEV)
        right = lax.rem(my + 1, N_DEV)

        barrier_sem = pltpu.get_barrier_semaphore()
        for nbr in (left, right):
            pl.semaphore_signal(
                barrier_sem, inc=1,
                device_id=(nbr,), device_id_type=pl.DeviceIdType.MESH,
            )
        pl.semaphore_wait(barrier_sem, 2)

        w16_ref[...] = w_ref[...].astype(jnp.bfloat16)

        def partial_half(c, col0):
            xs = x_ref[pl.ds(c * m_per, m_per), :].astype(jnp.bfloat16)
            return jnp.dot(xs, w16_ref[:, col0:col0 + n_half],
                           preferred_element_type=jnp.float32)

        comm_f[0, :, :] = partial_half(
            lax.rem(my + N_DEV - 1, N_DEV), 0).astype(jnp.bfloat16)
        comm_b[0, :, :] = partial_half(
            lax.rem(my + 1, N_DEV), n_half).astype(jnp.bfloat16)

        rdmas = []
        for h in range(N_DEV - 1):
            rdma_f = pltpu.make_async_remote_copy(
                src_ref=comm_f.at[h], dst_ref=comm_f.at[h + 1],
                send_sem=send_f.at[h], recv_sem=recv_f.at[h],
                device_id=(right,), device_id_type=pl.DeviceIdType.MESH,
            )
            rdma_b = pltpu.make_async_remote_copy(
                src_ref=comm_b.at[h], dst_ref=comm_b.at[h + 1],
                send_sem=send_b.at[h], recv_sem=recv_b.at[h],
                device_id=(left,), device_id_type=pl.DeviceIdType.MESH,
            )
            rdma_f.start()
            rdma_b.start()
            rdmas += [rdma_f, rdma_b]

            p_f = partial_half(lax.rem(my + 2 * N_DEV - 2 - h, N_DEV), 0)
            p_b = partial_half(lax.rem(my + 2 + h, N_DEV), n_half)

            rdma_f.wait_recv()
            rdma_b.wait_recv()
            if h < N_DEV - 2:
                acc_f = comm_f[h + 1, :, :].astype(jnp.float32) + p_f
                comm_f[h + 1, :, :] = acc_f.astype(jnp.bfloat16)
                acc_b = comm_b[h + 1, :, :].astype(jnp.float32) + p_b
                comm_b[h + 1, :, :] = acc_b.astype(jnp.bfloat16)
            else:
                y_f = comm_f[h + 1, :, :].astype(jnp.float32) + p_f
                out_ref[:, 0:n_half] = y_f * jax.nn.sigmoid(y_f)
                y_b = comm_b[h + 1, :, :].astype(jnp.float32) + p_b
                out_ref[:, n_half:n] = y_b * jax.nn.sigmoid(y_b)

        for r in rdmas:
            r.wait_send()

    return pl.pallas_call(
        body,
        out_shape=jax.ShapeDtypeStruct((m_per, n), jnp.float32),
        in_specs=[
            pl.BlockSpec(memory_space=pltpu.VMEM),
            pl.BlockSpec(memory_space=pltpu.VMEM),
        ],
        out_specs=pl.BlockSpec(memory_space=pltpu.VMEM),
        scratch_shapes=[
            pltpu.VMEM((N_DEV, m_per, n_half), jnp.bfloat16),
            pltpu.VMEM((N_DEV, m_per, n_half), jnp.bfloat16),
            pltpu.VMEM((kk, n), jnp.bfloat16),
            pltpu.SemaphoreType.DMA((N_DEV - 1,)),
            pltpu.SemaphoreType.DMA((N_DEV - 1,)),
            pltpu.SemaphoreType.DMA((N_DEV - 1,)),
            pltpu.SemaphoreType.DMA((N_DEV - 1,)),
        ],
        compiler_params=pltpu.CompilerParams(collective_id=0),
    )(x, w_mat)


# device time: 96086 ns/iter; 1.9623x vs baseline; 1.1735x over previous
import jax
import jax.numpy as jnp
from jax import lax
from jax.experimental import pallas as pl
from jax.experimental.pallas import tpu as pltpu

N_DEV = 8
NSUB = 2


def kernel(x, w_mat):
    m, _ = x.shape
    kk, n = w_mat.shape
    m_per = m // N_DEV
    n_half = n // 2
    m_sub = m_per // NSUB

    def body(x_ref, w_ref, out_ref, comm_f, comm_b, w16_ref,
             send_f, recv_f, send_b, recv_b):
        my = lax.axis_index("i")
        left = lax.rem(my + N_DEV - 1, N_DEV)
        right = lax.rem(my + 1, N_DEV)

        barrier_sem = pltpu.get_barrier_semaphore()
        for nbr in (left, right):
            pl.semaphore_signal(
                barrier_sem, inc=1,
                device_id=(nbr,), device_id_type=pl.DeviceIdType.MESH,
            )
        pl.semaphore_wait(barrier_sem, 2)

        w16_ref[...] = w_ref[...].astype(jnp.bfloat16)

        def partial_half(c, col0):
            xs = x_ref[pl.ds(c * m_per, m_per), :].astype(jnp.bfloat16)
            return jnp.dot(xs, w16_ref[:, col0:col0 + n_half],
                           preferred_element_type=jnp.float32)

        def c_fwd(k):
            return lax.rem(my + 2 * N_DEV - 1 - k, N_DEV)

        def c_bwd(k):
            return lax.rem(my + 1 + k, N_DEV)

        def mk(comm, ssem, rsem, hop, s, dev):
            return pltpu.make_async_remote_copy(
                src_ref=comm.at[hop, s],
                dst_ref=comm.at[hop + 1, s],
                send_sem=ssem.at[hop, s],
                recv_sem=rsem.at[hop, s],
                device_id=(dev,),
                device_id_type=pl.DeviceIdType.MESH,
            )

        p_f = partial_half(c_fwd(0), 0)
        comm_f[0, 0] = p_f[0:m_sub, :].astype(jnp.bfloat16)
        comm_f[0, 1] = p_f[m_sub:m_per, :].astype(jnp.bfloat16)
        p_b = partial_half(c_bwd(0), n_half)
        comm_b[0, 0] = p_b[0:m_sub, :].astype(jnp.bfloat16)
        comm_b[0, 1] = p_b[m_sub:m_per, :].astype(jnp.bfloat16)
        for s in range(NSUB):
            mk(comm_f, send_f, recv_f, 0, s, right).start()
            mk(comm_b, send_b, recv_b, 0, s, left).start()

        p_f = partial_half(c_fwd(1), 0)
        p_b = partial_half(c_bwd(1), n_half)

        for k in range(1, N_DEV):
            last = k == N_DEV - 1
            for comm, ssem, rsem, dev, p, col0 in (
                (comm_f, send_f, recv_f, right, p_f, 0),
                (comm_b, send_b, recv_b, left, p_b, n_half),
            ):
                for s in range(NSUB):
                    rows = slice(s * m_sub, (s + 1) * m_sub)
                    mk(comm, ssem, rsem, k - 1, s, dev).wait_recv()
                    acc = comm[k, s].astype(jnp.float32) + p[rows, :]
                    if not last:
                        comm[k, s] = acc.astype(jnp.bfloat16)
                        mk(comm, ssem, rsem, k, s, dev).start()
                    else:
                        out_ref[rows, col0:col0 + n_half] = (
                            acc * jax.nn.sigmoid(acc))
            if k < N_DEV - 1:
                p_f = partial_half(c_fwd(k + 1), 0)
                p_b = partial_half(c_bwd(k + 1), n_half)

        for hop in range(N_DEV - 1):
            for s in range(NSUB):
                mk(comm_f, send_f, recv_f, hop, s, right).wait_send()
                mk(comm_b, send_b, recv_b, hop, s, left).wait_send()

    return pl.pallas_call(
        body,
        out_shape=jax.ShapeDtypeStruct((m_per, n), jnp.float32),
        in_specs=[
            pl.BlockSpec(memory_space=pltpu.VMEM),
            pl.BlockSpec(memory_space=pltpu.VMEM),
        ],
        out_specs=pl.BlockSpec(memory_space=pltpu.VMEM),
        scratch_shapes=[
            pltpu.VMEM((N_DEV, NSUB, m_sub, n_half), jnp.bfloat16),
            pltpu.VMEM((N_DEV, NSUB, m_sub, n_half), jnp.bfloat16),
            pltpu.VMEM((kk, n), jnp.bfloat16),
            pltpu.SemaphoreType.DMA((N_DEV - 1, NSUB)),
            pltpu.SemaphoreType.DMA((N_DEV - 1, NSUB)),
            pltpu.SemaphoreType.DMA((N_DEV - 1, NSUB)),
            pltpu.SemaphoreType.DMA((N_DEV - 1, NSUB)),
        ],
        compiler_params=pltpu.CompilerParams(collective_id=0),
    )(x, w_mat)
